# baseline (speedup 1.0000x reference)
import functools

import numpy as np
import jax
import jax.numpy as jnp
from jax import lax
from jax.experimental import pallas as pl
from jax.experimental.pallas import tpu as pltpu
from jax.experimental.pallas import tpu_sc as plsc

_VOCAB = 100000
_DIM = 128
_SEQ = 200
_BATCH = 4096

_NC = 2
_NS = 16
_NW = _NC * _NS
_ROWS = _BATCH * _SEQ
_RPW = _ROWS // _NW
_C = 80
_NCHUNK = _RPW // _C
_PE2R = 240


def _pos_encoding() -> np.ndarray:
    pos = np.arange(_SEQ, dtype=np.float32)[:, None]
    div = np.exp(np.arange(0, _DIM, 2, dtype=np.float32) * (-np.log(10000.0) / _DIM))
    pe = np.zeros((_SEQ, _DIM), dtype=np.float32)
    pe[:, 0::2] = np.sin(pos * div)
    pe[:, 1::2] = np.cos(pos * div)
    return pe


_PE = _pos_encoding()


def _make_sc_kernel():
    mesh = plsc.VectorSubcoreMesh(core_axis_name="c", subcore_axis_name="s")

    @functools.partial(
        pl.kernel,
        mesh=mesh,
        out_type=jax.ShapeDtypeStruct((_ROWS, _DIM), jnp.float32),
        scratch_types=[
            pltpu.VMEM((8, _C), jnp.int32),
            pltpu.VMEM((_C, _DIM), jnp.float32),
            pltpu.VMEM((_C, _DIM), jnp.float32),
            pltpu.VMEM((_C, _DIM), jnp.float32),
            pltpu.VMEM((_C, _DIM), jnp.float32),
            pltpu.VMEM((_C, _DIM), jnp.float32),
            pltpu.VMEM((_C, _DIM), jnp.float32),
            pltpu.VMEM((_C, _DIM), jnp.float32),
            pltpu.VMEM((_C, _DIM), jnp.float32),
            pltpu.VMEM((_PE2R, _DIM), jnp.float32),
            pltpu.SemaphoreType.DMA,
            pltpu.SemaphoreType.DMA,
            pltpu.SemaphoreType.DMA,
            pltpu.SemaphoreType.DMA,
            pltpu.SemaphoreType.DMA,
            pltpu.SemaphoreType.DMA,
            pltpu.SemaphoreType.DMA,
            pltpu.SemaphoreType.DMA,
            pltpu.SemaphoreType.DMA,
            pltpu.SemaphoreType.DMA,
            pltpu.SemaphoreType.DMA,
            pltpu.SemaphoreType.DMA,
            pltpu.SemaphoreType.DMA,
            pltpu.SemaphoreType.DMA,
            pltpu.SemaphoreType.DMA,
            pltpu.SemaphoreType.DMA,
        ],
    )
    def body(xf, table, pe, out, idx_v, rows0, rows1, rows2, rows3,
             st0, st1, st2, st3, pe_v, gsem0, gsem1, gsem2, gsem3,
             osem0, osem1, osem2, osem3,
             isem0, isem1, isem2, isem3, isem4, isem5, isem6, isem7):
        wid = lax.axis_index("s") * _NC + lax.axis_index("c")
        base = wid * _RPW
        rows_b = (rows0, rows1, rows2, rows3)
        st_b = (st0, st1, st2, st3)
        gsem = (gsem0, gsem1, gsem2, gsem3)
        osem = (osem0, osem1, osem2, osem3)
        isem = (isem0, isem1, isem2, isem3, isem4, isem5, isem6, isem7)

        pltpu.sync_copy(pe, pe_v)

        def fire_idx(g, ib):
            pltpu.async_copy(xf.at[wid, g], idx_v.at[ib], isem[ib])

        def fire(g, gb, ib):
            pltpu.make_async_copy(xf.at[wid, g], idx_v.at[ib], isem[ib]).wait()
            pltpu.async_copy(table.at[idx_v.at[ib]], rows_b[gb], gsem[gb])

        for _g in range(8):
            fire_idx(_g, _g)
        for _g in range(4):
            fire(_g, _g, _g)

        def compute(g, gb, ob):
            src = rows_b[gb]
            dst = st_b[ob]
            s_start = lax.rem(g * _C, _SEQ)

            @plsc.parallel_loop(0, _C, unroll=8)
            def row(r):
                s = s_start + r
                for d in range(_DIM // 16):
                    sl = pl.ds(d * 16, 16)
                    e = src[r, sl]
                    dst[r, sl] = e + e + pe_v[s, sl]

        def step(g, gb, ob, ib):
            pltpu.make_async_copy(table.at[idx_v.at[ib]], rows_b[gb], gsem[gb]).wait()

            @pl.when(g + 8 < _NCHUNK)
            def _fire_next_idx():
                fire_idx(g + 8, ib)

            @pl.when(g >= 4)
            def _wait_prev_out():
                pltpu.make_async_copy(
                    st_b[ob], out.at[pl.ds(base + (g - 4) * _C, _C)], osem[ob]
                ).wait()

            compute(g, gb, ob)
            pltpu.async_copy(st_b[ob], out.at[pl.ds(base + g * _C, _C)], osem[ob])

            @pl.when(g + 4 < _NCHUNK)
            def _fire_next():
                fire(g + 4, gb, (ib + 4) % 8)

        def oct_(i, carry):
            for b in range(8):
                step(8 * i + b, b % 4, b % 4, b)
            return carry

        lax.fori_loop(0, _NCHUNK // 8, oct_, None)

        pltpu.make_async_copy(
            st0, out.at[pl.ds(base + (_NCHUNK - 4) * _C, _C)], osem0).wait()
        pltpu.make_async_copy(
            st1, out.at[pl.ds(base + (_NCHUNK - 3) * _C, _C)], osem1).wait()
        pltpu.make_async_copy(
            st2, out.at[pl.ds(base + (_NCHUNK - 2) * _C, _C)], osem2).wait()
        pltpu.make_async_copy(
            st3, out.at[pl.ds(base + (_NCHUNK - 1) * _C, _C)], osem3).wait()

    return body


_PE2 = np.concatenate([_PE, _PE[: _PE2R - _SEQ]], axis=0)


def kernel(x, table):
    xf = x.reshape(_NW, _NCHUNK, _C)
    pe = jnp.asarray(_PE2)
    out = _make_sc_kernel()(xf, table, pe)
    return out.reshape(_BATCH, _SEQ, _DIM)


# speedup vs baseline: 1.0259x; 1.0259x over previous
import functools

import numpy as np
import jax
import jax.numpy as jnp
from jax import lax
from jax.experimental import pallas as pl
from jax.experimental.pallas import tpu as pltpu
from jax.experimental.pallas import tpu_sc as plsc

_VOCAB = 100000
_DIM = 128
_SEQ = 200
_BATCH = 4096

_NC = 2
_NS = 16
_NW = _NC * _NS
_ROWS = _BATCH * _SEQ
_RPW = _ROWS // _NW
_C = 80
_NCHUNK = _RPW // _C
_PE2R = 240


def _pos_encoding() -> np.ndarray:
    pos = np.arange(_SEQ, dtype=np.float32)[:, None]
    div = np.exp(np.arange(0, _DIM, 2, dtype=np.float32) * (-np.log(10000.0) / _DIM))
    pe = np.zeros((_SEQ, _DIM), dtype=np.float32)
    pe[:, 0::2] = np.sin(pos * div)
    pe[:, 1::2] = np.cos(pos * div)
    return pe


_PE = _pos_encoding()


def _make_sc_kernel():
    mesh = plsc.VectorSubcoreMesh(core_axis_name="c", subcore_axis_name="s")

    @functools.partial(
        pl.kernel,
        mesh=mesh,
        out_type=jax.ShapeDtypeStruct((_ROWS, _DIM), jnp.float32),
        scratch_types=[
            pltpu.VMEM((8, _C), jnp.int32),
            pltpu.VMEM((_C, _DIM), jnp.float32),
            pltpu.VMEM((_C, _DIM), jnp.float32),
            pltpu.VMEM((_C, _DIM), jnp.float32),
            pltpu.VMEM((_C, _DIM), jnp.float32),
            pltpu.VMEM((_C, _DIM), jnp.float32),
            pltpu.VMEM((_C, _DIM), jnp.float32),
            pltpu.VMEM((_C, _DIM), jnp.float32),
            pltpu.VMEM((_C, _DIM), jnp.float32),
            pltpu.VMEM((_PE2R, _DIM), jnp.float32),
            pltpu.SemaphoreType.DMA,
            pltpu.SemaphoreType.DMA,
            pltpu.SemaphoreType.DMA,
            pltpu.SemaphoreType.DMA,
            pltpu.SemaphoreType.DMA,
            pltpu.SemaphoreType.DMA,
            pltpu.SemaphoreType.DMA,
            pltpu.SemaphoreType.DMA,
            pltpu.SemaphoreType.DMA,
            pltpu.SemaphoreType.DMA,
            pltpu.SemaphoreType.DMA,
            pltpu.SemaphoreType.DMA,
            pltpu.SemaphoreType.DMA,
            pltpu.SemaphoreType.DMA,
            pltpu.SemaphoreType.DMA,
            pltpu.SemaphoreType.DMA,
        ],
    )
    def body(xf, table, pe, out, idx_v, rows0, rows1, rows2, rows3,
             st0, st1, st2, st3, pe_v, gsem0, gsem1, gsem2, gsem3,
             osem0, osem1, osem2, osem3,
             isem0, isem1, isem2, isem3, isem4, isem5, isem6, isem7):
        wid = lax.axis_index("s") * _NC + lax.axis_index("c")
        base = wid * _RPW
        rows_b = (rows0, rows1, rows2, rows3)
        st_b = (st0, st1, st2, st3)
        gsem = (gsem0, gsem1, gsem2, gsem3)
        osem = (osem0, osem1, osem2, osem3)
        isem = (isem0, isem1, isem2, isem3, isem4, isem5, isem6, isem7)

        pltpu.sync_copy(pe, pe_v)

        def fire_idx(g, ib):
            pltpu.async_copy(xf.at[wid, g], idx_v.at[ib], isem[ib])

        def fire(g, gb, ib):
            pltpu.make_async_copy(xf.at[wid, g], idx_v.at[ib], isem[ib]).wait()
            pltpu.async_copy(table.at[idx_v.at[ib]], rows_b[gb], gsem[gb])

        for _g in range(8):
            fire_idx(_g, _g)
        for _g in range(4):
            fire(_g, _g, _g)

        def compute(g, gb, ob):
            src = rows_b[gb]
            dst = st_b[ob]
            s_start = lax.rem(g * _C, _SEQ)

            @plsc.parallel_loop(0, _C, unroll=8)
            def row(r):
                s = s_start + r
                for d in range(_DIM // 16):
                    sl = pl.ds(d * 16, 16)
                    e = src[r, sl]
                    dst[r, sl] = e + e + pe_v[s, sl]

        def step(g, gb, ob, ib):
            pltpu.make_async_copy(table.at[idx_v.at[ib]], rows_b[gb], gsem[gb]).wait()

            @pl.when(g + 8 < _NCHUNK)
            def _fire_next_idx():
                fire_idx(g + 8, ib)

            @pl.when(g >= 4)
            def _wait_prev_out():
                pltpu.make_async_copy(
                    st_b[ob], out.at[pl.ds(base + (g - 4) * _C, _C)], osem[ob]
                ).wait()

            pltpu.async_copy(st_b[ob], out.at[pl.ds(base + g * _C, _C)], osem[ob])

            @pl.when(g + 4 < _NCHUNK)
            def _fire_next():
                fire(g + 4, gb, (ib + 4) % 8)

        def oct_(i, carry):
            for b in range(8):
                step(8 * i + b, b % 4, b % 4, b)
            return carry

        lax.fori_loop(0, _NCHUNK // 8, oct_, None)

        pltpu.make_async_copy(
            st0, out.at[pl.ds(base + (_NCHUNK - 4) * _C, _C)], osem0).wait()
        pltpu.make_async_copy(
            st1, out.at[pl.ds(base + (_NCHUNK - 3) * _C, _C)], osem1).wait()
        pltpu.make_async_copy(
            st2, out.at[pl.ds(base + (_NCHUNK - 2) * _C, _C)], osem2).wait()
        pltpu.make_async_copy(
            st3, out.at[pl.ds(base + (_NCHUNK - 1) * _C, _C)], osem3).wait()

    return body


_PE2 = np.concatenate([_PE, _PE[: _PE2R - _SEQ]], axis=0)


def kernel(x, table):
    xf = x.reshape(_NW, _NCHUNK, _C)
    pe = jnp.asarray(_PE2)
    out = _make_sc_kernel()(xf, table, pe)
    return out.reshape(_BATCH, _SEQ, _DIM)
